# baseline (speedup 1.0000x reference)
import functools

import jax
import jax.numpy as jnp
from jax import lax
from jax.experimental import pallas as pl
from jax.experimental.pallas import tpu as pltpu
from jax.experimental.pallas import tpu_sc as plsc

_B, _N, _F, _S, _OUT = 4, 10000, 128, 32, 128
_GBLK = 128
_NPAD = 10048
_CBLK_PAD = _S * _NPAD // _GBLK
_IDX_PAD = 2520
_MPAD = _CBLK_PAD * _GBLK
_NQ = 4
_BLK = 1000


@functools.lru_cache(maxsize=None)
def _make_sc_gather():
    info = plsc.get_sparse_core_info()
    nc, ns = info.num_cores, info.num_subcores
    nw = nc * ns
    noct = _CBLK_PAD // 8
    base_o = noct // nw
    rem = noct - base_o * nw
    slab = 8 * (base_o + 1)

    mesh = plsc.VectorSubcoreMesh(core_axis_name="c", subcore_axis_name="s")

    @functools.partial(
        pl.kernel,
        mesh=mesh,
        out_type=jax.ShapeDtypeStruct((_MPAD, _F), jnp.float32),
        scratch_types=[
            pltpu.VMEM((slab, _GBLK), jnp.int32),
            pltpu.VMEM((_NQ, _GBLK, _F), jnp.float32),
            pltpu.SemaphoreType.DMA((_NQ,)),
            pltpu.SemaphoreType.DMA((_NQ,)),
        ],
    )
    def sc_gather(table_hbm, idx_hbm, out_hbm, idx_slab, rows_v,
                  gsem, ssem):
        cid = lax.axis_index("c")
        sid = lax.axis_index("s")
        wid = sid * nc + cid
        ostart = wid * base_o + jnp.minimum(wid, rem)
        nquads = (8 // _NQ) * (base_o + (wid < rem).astype(jnp.int32))
        blk0 = ostart * 8

        pltpu.sync_copy(idx_hbm.at[pl.ds(blk0, slab)], idx_slab)

        def body(t, carry):
            @pl.when(t > 0)
            def _():
                for k in range(_NQ):
                    pltpu.make_async_copy(
                        rows_v.at[k], out_hbm.at[pl.ds(0, _GBLK)], ssem.at[k]
                    ).wait()

            handles = [
                pltpu.async_copy(
                    table_hbm.at[idx_slab.at[_NQ * t + k]],
                    rows_v.at[k],
                    gsem.at[k],
                )
                for k in range(_NQ)
            ]
            for k in range(_NQ):
                handles[k].wait()
                pltpu.async_copy(
                    rows_v.at[k],
                    out_hbm.at[pl.ds((blk0 + _NQ * t + k) * _GBLK, _GBLK)],
                    ssem.at[k],
                )
            return carry

        lax.fori_loop(0, nquads, body, 0, unroll=False)

        for k in range(_NQ):
            pltpu.make_async_copy(
                rows_v.at[k], out_hbm.at[pl.ds(0, _GBLK)], ssem.at[k]
            ).wait()

    return sc_gather


def _mm_body(g_ref, w_ref, b_ref, o_ref):
    acc = jnp.zeros((_BLK, _OUT), jnp.float32)
    for s in range(_S):
        acc += jnp.dot(
            g_ref[s].astype(jnp.bfloat16),
            w_ref[s],
            preferred_element_type=jnp.float32,
        )
    y = acc + b_ref[...]
    y = jnp.where(y > 0, y, jnp.exp(jnp.minimum(y, 0.0)) - 1.0)
    i = pl.program_id(0)
    rows = i * _BLK + lax.broadcasted_iota(jnp.int32, (_BLK, 1), 0)
    o_ref[...] = jnp.where(rows == _N - 1, 0.0, y)


def _tc_matmul(g3, w3, bias):
    return pl.pallas_call(
        _mm_body,
        grid=(_N // _BLK,),
        in_specs=[
            pl.BlockSpec((_S, _BLK, _F), lambda i: (0, i, 0)),
            pl.BlockSpec((_S, _F, _OUT), lambda i: (0, 0, 0)),
            pl.BlockSpec((1, _OUT), lambda i: (0, 0)),
        ],
        out_specs=pl.BlockSpec((_BLK, _OUT), lambda i: (i, 0)),
        out_shape=jax.ShapeDtypeStruct((_N, _OUT), jnp.float32),
        compiler_params=pltpu.CompilerParams(
            dimension_semantics=("arbitrary",),
        ),
    )(g3, w3, bias)


def kernel(x, spiral_adj, W, b):
    adjp = jnp.pad(
        spiral_adj.transpose(0, 2, 1),
        ((0, 0), (0, 0), (0, _NPAD - _N)),
    ).reshape(_B, _CBLK_PAD, _GBLK)
    adjp = jnp.pad(adjp, ((0, 0), (0, _IDX_PAD - _CBLK_PAD), (0, 0)))

    w3 = W.reshape(_S, _F, _OUT).astype(jnp.bfloat16)
    bias = b.reshape(1, _OUT)
    gather = _make_sc_gather()

    outs = []
    for bb in range(_B):
        gathered = gather(x[bb], adjp[bb])
        g3 = gathered.reshape(_S, _NPAD, _F)
        outs.append(_tc_matmul(g3, w3, bias))
    return jnp.stack(outs)


# speedup vs baseline: 1.9255x; 1.9255x over previous
import functools

import jax
import jax.numpy as jnp
from jax import lax
from jax.experimental import pallas as pl
from jax.experimental.pallas import tpu as pltpu
from jax.experimental.pallas import tpu_sc as plsc

_B, _N, _F, _S, _OUT = 4, 10000, 128, 32, 128
_GBLK = 128
_NPAD = 10048
_CBLK_PAD = _S * _NPAD // _GBLK
_IDX_PAD = 2520
_MPAD = _CBLK_PAD * _GBLK
_NQ = 2
_BLK = 1000


@functools.lru_cache(maxsize=None)
def _make_sc_gather():
    info = plsc.get_sparse_core_info()
    nc, ns = info.num_cores, info.num_subcores
    nw = nc * ns
    noct = _CBLK_PAD // 8
    base_o = noct // nw
    rem = noct - base_o * nw
    slab = 8 * (base_o + 1)

    mesh = plsc.VectorSubcoreMesh(core_axis_name="c", subcore_axis_name="s")

    @functools.partial(
        pl.kernel,
        mesh=mesh,
        out_type=jax.ShapeDtypeStruct((_MPAD, _F), jnp.float32),
        scratch_types=[
            pltpu.VMEM_SHARED((_N, _F), jnp.float32),
            pltpu.VMEM((slab, _GBLK), jnp.int32),
            pltpu.VMEM((_NQ, _GBLK, _F), jnp.float32),
            pltpu.SemaphoreType.DMA((_NQ,)),
            pltpu.SemaphoreType.DMA((_NQ,)),
        ],
    )
    def sc_gather(table_hbm, idx_hbm, out_hbm, table_sp, idx_slab, rows_v,
                  gsem, ssem):
        cid = lax.axis_index("c")
        sid = lax.axis_index("s")
        wid = sid * nc + cid
        ostart = wid * base_o + jnp.minimum(wid, rem)
        nquads = (8 // _NQ) * (base_o + (wid < rem).astype(jnp.int32))
        blk0 = ostart * 8

        pltpu.sync_copy(idx_hbm.at[pl.ds(blk0, slab)], idx_slab)

        @pl.when(sid == 0)
        def _():
            pltpu.sync_copy(table_hbm, table_sp)

        plsc.subcore_barrier()

        def body(t, carry):
            handles = []
            for k in range(_NQ):
                @pl.when(t > 0)
                def _(k=k):
                    pltpu.make_async_copy(
                        rows_v.at[k], out_hbm.at[pl.ds(0, _GBLK)], ssem.at[k]
                    ).wait()

                handles.append(
                    pltpu.async_copy(
                        table_sp.at[idx_slab.at[_NQ * t + k]],
                        rows_v.at[k],
                        gsem.at[k],
                    )
                )
            for k in range(_NQ):
                handles[k].wait()
                pltpu.async_copy(
                    rows_v.at[k],
                    out_hbm.at[pl.ds((blk0 + _NQ * t + k) * _GBLK, _GBLK)],
                    ssem.at[k],
                )
            return carry

        lax.fori_loop(0, nquads, body, 0, unroll=False)

        for k in range(_NQ):
            pltpu.make_async_copy(
                rows_v.at[k], out_hbm.at[pl.ds(0, _GBLK)], ssem.at[k]
            ).wait()

    return sc_gather


_NH = _N // 2


def _mm_body(h, g_ref, w_ref, b_ref, o_ref):
    acc = jnp.zeros((_BLK, _OUT), jnp.float32)
    for s in range(_S):
        acc += jnp.dot(
            g_ref[s].astype(jnp.bfloat16),
            w_ref[s],
            preferred_element_type=jnp.float32,
        )
    y = acc + b_ref[...]
    y = jnp.where(y > 0, y, jnp.exp(jnp.minimum(y, 0.0)) - 1.0)
    i = pl.program_id(0)
    rows = h * _NH + i * _BLK + lax.broadcasted_iota(jnp.int32, (_BLK, 1), 0)
    o_ref[...] = jnp.where(rows == _N - 1, 0.0, y)


def _tc_matmul(g3, w3, bias, h):
    nb = _NH // _BLK
    return pl.pallas_call(
        functools.partial(_mm_body, h),
        grid=(nb,),
        in_specs=[
            pl.BlockSpec((_S, _BLK, _F), lambda i: (0, i + h * nb, 0)),
            pl.BlockSpec((_S, _F, _OUT), lambda i: (0, 0, 0)),
            pl.BlockSpec((1, _OUT), lambda i: (0, 0)),
        ],
        out_specs=pl.BlockSpec((_BLK, _OUT), lambda i: (i, 0)),
        out_shape=jax.ShapeDtypeStruct((_NH, _OUT), jnp.float32),
        compiler_params=pltpu.CompilerParams(
            dimension_semantics=("arbitrary",),
        ),
    )(g3, w3, bias)


def kernel(x, spiral_adj, W, b):
    adjp = jnp.pad(
        spiral_adj.transpose(0, 2, 1),
        ((0, 0), (0, 0), (0, _NPAD - _N)),
    ).reshape(_B, _CBLK_PAD, _GBLK)
    adjp = jnp.pad(adjp, ((0, 0), (0, _IDX_PAD - _CBLK_PAD), (0, 0)))

    w3 = W.reshape(_S, _F, _OUT).astype(jnp.bfloat16)
    bias = b.reshape(1, _OUT)
    gather = _make_sc_gather()

    outs = []
    for bb in range(_B):
        gathered = gather(x[bb], adjp[bb])
        g3 = gathered.reshape(_S, _NPAD, _F)
        outs.append(_tc_matmul(g3, w3, bias, 0))
        outs.append(_tc_matmul(g3, w3, bias, 1))
    return jnp.concatenate(outs).reshape(_B, _N, _OUT)


# speedup vs baseline: 1.9421x; 1.0086x over previous
import functools

import jax
import jax.numpy as jnp
from jax import lax
from jax.experimental import pallas as pl
from jax.experimental.pallas import tpu as pltpu
from jax.experimental.pallas import tpu_sc as plsc

_B, _N, _F, _S, _OUT = 4, 10000, 128, 32, 128
_GBLK = 128
_NPAD = 10048
_CBLK_PAD = _S * _NPAD // _GBLK
_IDX_PAD = 2520
_MPAD = _CBLK_PAD * _GBLK
_NQ = 2
_BLK = 1000


@functools.lru_cache(maxsize=None)
def _make_sc_gather():
    info = plsc.get_sparse_core_info()
    nc, ns = info.num_cores, info.num_subcores
    nw = nc * ns
    noct = _CBLK_PAD // 8
    base_o = noct // nw
    rem = noct - base_o * nw
    slab = 8 * (base_o + 1)

    mesh = plsc.VectorSubcoreMesh(core_axis_name="c", subcore_axis_name="s")

    @functools.partial(
        pl.kernel,
        mesh=mesh,
        out_type=jax.ShapeDtypeStruct((_MPAD, _F), jnp.float32),
        scratch_types=[
            pltpu.VMEM_SHARED((_N, _F), jnp.float32),
            pltpu.VMEM((slab, _GBLK), jnp.int32),
            pltpu.VMEM((_NQ, _GBLK, _F), jnp.float32),
            pltpu.SemaphoreType.DMA((_NQ,)),
            pltpu.SemaphoreType.DMA((_NQ,)),
        ],
    )
    def sc_gather(table_hbm, idx_hbm, out_hbm, table_sp, idx_slab, rows_v,
                  gsem, ssem):
        cid = lax.axis_index("c")
        sid = lax.axis_index("s")
        wid = sid * nc + cid
        ostart = wid * base_o + jnp.minimum(wid, rem)
        nquads = (8 // _NQ) * (base_o + (wid < rem).astype(jnp.int32))
        blk0 = ostart * 8

        pltpu.sync_copy(idx_hbm.at[pl.ds(blk0, slab)], idx_slab)

        @pl.when(sid == 0)
        def _():
            pltpu.sync_copy(table_hbm, table_sp)

        plsc.subcore_barrier()

        def body(t, carry):
            @pl.when(t > 0)
            def _():
                for k in range(_NQ):
                    pltpu.make_async_copy(
                        rows_v.at[k], out_hbm.at[pl.ds(0, _GBLK)], ssem.at[k]
                    ).wait()

            handles = [
                pltpu.async_copy(
                    table_sp.at[idx_slab.at[_NQ * t + k]],
                    rows_v.at[k],
                    gsem.at[k],
                )
                for k in range(_NQ)
            ]
            for k in range(_NQ):
                handles[k].wait()
                pltpu.async_copy(
                    rows_v.at[k],
                    out_hbm.at[pl.ds((blk0 + _NQ * t + k) * _GBLK, _GBLK)],
                    ssem.at[k],
                )
            return carry

        lax.fori_loop(0, nquads, body, 0, unroll=False)

        for k in range(_NQ):
            pltpu.make_async_copy(
                rows_v.at[k], out_hbm.at[pl.ds(0, _GBLK)], ssem.at[k]
            ).wait()

    return sc_gather


def _mm_body(g_ref, w_ref, b_ref, o_ref):
    acc = jnp.zeros((_BLK, _OUT), jnp.float32)
    for s in range(_S):
        acc += jnp.dot(
            g_ref[s].astype(jnp.bfloat16),
            w_ref[s],
            preferred_element_type=jnp.float32,
        )
    y = acc + b_ref[...]
    y = jnp.where(y > 0, y, jnp.exp(jnp.minimum(y, 0.0)) - 1.0)
    i = pl.program_id(0)
    rows = i * _BLK + lax.broadcasted_iota(jnp.int32, (_BLK, 1), 0)
    o_ref[...] = jnp.where(rows == _N - 1, 0.0, y)


def _tc_matmul(g3, w3, bias):
    return pl.pallas_call(
        _mm_body,
        grid=(_N // _BLK,),
        in_specs=[
            pl.BlockSpec((_S, _BLK, _F), lambda i: (0, i, 0)),
            pl.BlockSpec((_S, _F, _OUT), lambda i: (0, 0, 0)),
            pl.BlockSpec((1, _OUT), lambda i: (0, 0)),
        ],
        out_specs=pl.BlockSpec((_BLK, _OUT), lambda i: (i, 0)),
        out_shape=jax.ShapeDtypeStruct((_N, _OUT), jnp.float32),
        compiler_params=pltpu.CompilerParams(
            dimension_semantics=("arbitrary",),
        ),
    )(g3, w3, bias)


def kernel(x, spiral_adj, W, b):
    adjp = jnp.pad(
        spiral_adj.transpose(0, 2, 1),
        ((0, 0), (0, 0), (0, _NPAD - _N)),
    ).reshape(_B, _CBLK_PAD, _GBLK)
    adjp = jnp.pad(adjp, ((0, 0), (0, _IDX_PAD - _CBLK_PAD), (0, 0)))

    w3 = W.reshape(_S, _F, _OUT).astype(jnp.bfloat16)
    bias = b.reshape(1, _OUT)
    gather = _make_sc_gather()

    outs = []
    for bb in range(_B):
        gathered = gather(x[bb], adjp[bb])
        g3 = gathered.reshape(_S, _NPAD, _F)
        outs.append(_tc_matmul(g3, w3, bias))
    return jnp.stack(outs)
